# baseline (speedup 1.0000x reference)
import jax
import jax.numpy as jnp
from jax import lax
from jax.experimental import pallas as pl
from jax.experimental.pallas import tpu as pltpu
from jax.experimental.pallas import tpu_sc as plsc

VOCAB = 50257
HIDDEN = 1024
BATCH = 4
SEQ = 8192

_info = plsc.get_sparse_core_info()
NC, NS = _info.num_cores, _info.num_subcores
NW = NC * NS
POS_PER_W = SEQ // NW
PC = 16
NPC = POS_PER_W // PC
NSTEP = NPC * 2
NRB = 3
LANES = 16
IDX_PER_W = POS_PER_W * BATCH


def _body(idsr_hbm, tok_hbm, pos_hbm, out_hbm,
          idx_v, pos_v, rows_b, gsems, osems, psem):
    wid = lax.axis_index("s") * NC + lax.axis_index("c")
    s0 = wid * POS_PER_W

    pltpu.sync_copy(idsr_hbm.at[pl.ds(wid * IDX_PER_W, IDX_PER_W)], idx_v)

    def gather_cp(pc, bp, rb):
        off = pc * (2 * PC * 2) + bp * (2 * PC)
        return pltpu.make_async_copy(
            tok_hbm.at[idx_v.at[pl.ds(off, 2 * PC)]],
            rows_b.at[rb], gsems.at[rb])

    def out_cp(pc, bp, rb, h):
        return pltpu.make_async_copy(
            rows_b.at[rb, pl.ds(h * PC, PC)],
            out_hbm.at[pl.ds((2 * bp + h) * SEQ + s0 + pc * PC, PC)],
            osems.at[rb])

    def pos_cp(pc):
        return pltpu.make_async_copy(
            pos_hbm.at[pl.ds(s0 + pc * PC, PC)], pos_v, psem)

    def add_rows(rb):
        @plsc.parallel_loop(0, PC, unroll=4)
        def _(r):
            for j in range(HIDDEN // LANES):
                sl = pl.ds(j * LANES, LANES)
                x = pos_v[r, sl]
                plsc.addupdate(rows_b.at[rb, r, sl], x)
                plsc.addupdate(rows_b.at[rb, PC + r, sl], x)

    pos_cp(0).start()
    gather_cp(0, 0, 0).start()

    def cps(t):
        pc = lax.div(t, 2)
        bp = lax.rem(t, 2)
        rb = lax.rem(t, NRB)
        return pc, bp, rb

    def step(t, carry):
        pc, bp, rb = cps(t)
        @pl.when(t >= 2)
        def _():
            pc2, bp2, rb2 = cps(t - 2)
            for h in range(2):
                out_cp(pc2, bp2, rb2, h).wait()

        @pl.when(t + 1 < NSTEP)
        def _():
            pc1, bp1, rb1 = cps(t + 1)
            gather_cp(pc1, bp1, rb1).start()

        @pl.when(bp == 0)
        def _():
            pos_cp(pc).wait()

        gather_cp(pc, bp, rb).wait()
        add_rows(rb)

        @pl.when((bp == 1) & (pc + 1 < NPC))
        def _():
            pos_cp(pc + 1).start()

        for h in range(2):
            out_cp(pc, bp, rb, h).start()
        return carry

    lax.fori_loop(0, NSTEP, step, 0)

    for t in (NSTEP - 2, NSTEP - 1):
        pc, bp, rb = t // 2, t % 2, t % NRB
        for h in range(2):
            out_cp(pc, bp, rb, h).wait()


@jax.jit
def _embed(ids_re, token_table, pos_table):
    mesh = plsc.VectorSubcoreMesh(core_axis_name="c", subcore_axis_name="s")
    k = pl.kernel(
        _body,
        out_type=jax.ShapeDtypeStruct((BATCH * SEQ, HIDDEN), jnp.float32),
        mesh=mesh,
        scratch_types=[
            pltpu.VMEM((IDX_PER_W,), jnp.int32),
            pltpu.VMEM((PC, HIDDEN), jnp.float32),
            pltpu.VMEM((NRB, 2 * PC, HIDDEN), jnp.float32),
            pltpu.SemaphoreType.DMA((NRB,)),
            pltpu.SemaphoreType.DMA((NRB,)),
            pltpu.SemaphoreType.DMA,
        ],
    )
    return k(ids_re, token_table, pos_table)


def kernel(input_ids, token_table, pos_table):
    ids_re = (
        input_ids.astype(jnp.int32)
        .reshape(BATCH, SEQ // PC, PC)
        .transpose(1, 0, 2)
        .reshape(BATCH * SEQ)
    )
    out = _embed(ids_re, token_table, pos_table)
    return out.reshape(BATCH, SEQ, HIDDEN)


# speedup vs baseline: 1.1317x; 1.1317x over previous
import jax
import jax.numpy as jnp
from jax import lax
from jax.experimental import pallas as pl
from jax.experimental.pallas import tpu as pltpu
from jax.experimental.pallas import tpu_sc as plsc

VOCAB = 50257
HIDDEN = 1024
BATCH = 4
SEQ = 8192

_info = plsc.get_sparse_core_info()
NC, NS = _info.num_cores, _info.num_subcores
NW = NC * NS
POS_PER_W = SEQ // NW
PC = 16
NPC = POS_PER_W // PC
NSTEP = NPC * 2
NRB = 3
LANES = 16
IDX_PER_W = POS_PER_W * BATCH


def _body(idsr_hbm, tok_hbm, pos_hbm, out_hbm,
          idx_v, pos_v, rows_b, gsems, osems, psem):
    wid = lax.axis_index("s") * NC + lax.axis_index("c")
    s0 = wid * POS_PER_W

    pltpu.sync_copy(idsr_hbm.at[pl.ds(wid * IDX_PER_W, IDX_PER_W)], idx_v)

    def gather_cp(pc, bp, rb):
        off = pc * (2 * PC * 2) + bp * (2 * PC)
        return pltpu.make_async_copy(
            tok_hbm.at[idx_v.at[pl.ds(off, 2 * PC)]],
            rows_b.at[rb], gsems.at[rb])

    def out_cp(pc, bp, rb, h):
        return pltpu.make_async_copy(
            rows_b.at[rb, pl.ds(h * PC, PC)],
            out_hbm.at[pl.ds((2 * bp + h) * SEQ + s0 + pc * PC, PC)],
            osems.at[rb])

    HPC = PC // 2

    def pos_cp(pc, half):
        return pltpu.make_async_copy(
            pos_hbm.at[pl.ds(s0 + pc * PC + half * HPC, HPC)],
            pos_v.at[pl.ds(half * HPC, HPC)], psem.at[half])

    def add_rows(rb, half):
        @plsc.parallel_loop(half * HPC, (half + 1) * HPC, unroll=2)
        def _(r):
            for j in range(HIDDEN // LANES):
                sl = pl.ds(j * LANES, LANES)
                x = pos_v[r, sl]
                plsc.addupdate(rows_b.at[rb, r, sl], x)
                plsc.addupdate(rows_b.at[rb, PC + r, sl], x)

    pos_cp(0, 0).start()
    pos_cp(0, 1).start()
    gather_cp(0, 0, 0).start()

    def cps(t):
        pc = lax.div(t, 2)
        bp = lax.rem(t, 2)
        rb = lax.rem(t, NRB)
        return pc, bp, rb

    def step(t, carry):
        pc, bp, rb = cps(t)
        @pl.when(t >= 2)
        def _():
            pc2, bp2, rb2 = cps(t - 2)
            for h in range(2):
                out_cp(pc2, bp2, rb2, h).wait()

        @pl.when(t + 1 < NSTEP)
        def _():
            pc1, bp1, rb1 = cps(t + 1)
            gather_cp(pc1, bp1, rb1).start()

        gather_cp(pc, bp, rb).wait()

        @pl.when(bp == 0)
        def _():
            pos_cp(pc, 0).wait()

        add_rows(rb, 0)

        @pl.when(bp == 0)
        def _():
            pos_cp(pc, 1).wait()

        add_rows(rb, 1)

        @pl.when((bp == 1) & (pc + 1 < NPC))
        def _():
            pos_cp(pc + 1, 0).start()
            pos_cp(pc + 1, 1).start()

        for h in range(2):
            out_cp(pc, bp, rb, h).start()
        return carry

    lax.fori_loop(0, NSTEP, step, 0)

    for t in (NSTEP - 2, NSTEP - 1):
        pc, bp, rb = t // 2, t % 2, t % NRB
        for h in range(2):
            out_cp(pc, bp, rb, h).wait()


@jax.jit
def _embed(ids_re, token_table, pos_table):
    mesh = plsc.VectorSubcoreMesh(core_axis_name="c", subcore_axis_name="s")
    k = pl.kernel(
        _body,
        out_type=jax.ShapeDtypeStruct((BATCH * SEQ, HIDDEN), jnp.float32),
        mesh=mesh,
        scratch_types=[
            pltpu.VMEM((IDX_PER_W,), jnp.int32),
            pltpu.VMEM((PC, HIDDEN), jnp.float32),
            pltpu.VMEM((NRB, 2 * PC, HIDDEN), jnp.float32),
            pltpu.SemaphoreType.DMA((NRB,)),
            pltpu.SemaphoreType.DMA((NRB,)),
            pltpu.SemaphoreType.DMA((2,)),
        ],
    )
    return k(ids_re, token_table, pos_table)


def kernel(input_ids, token_table, pos_table):
    ids_re = (
        input_ids.astype(jnp.int32)
        .reshape(BATCH, SEQ // PC, PC)
        .transpose(1, 0, 2)
        .reshape(BATCH * SEQ)
    )
    out = _embed(ids_re, token_table, pos_table)
    return out.reshape(BATCH, SEQ, HIDDEN)


# speedup vs baseline: 1.6011x; 1.4148x over previous
import jax
import jax.numpy as jnp
from jax import lax
from jax.experimental import pallas as pl
from jax.experimental.pallas import tpu as pltpu
from jax.experimental.pallas import tpu_sc as plsc

VOCAB = 50257
HIDDEN = 1024
BATCH = 4
SEQ = 8192

_info = plsc.get_sparse_core_info()
NC, NS = _info.num_cores, _info.num_subcores
NW = NC * NS
POS_PER_W = SEQ // NW
PC = 16
NPC = POS_PER_W // PC
NSTEP = NPC * 2
NRB = 3
LANES = 16
IDX_PER_W = POS_PER_W * BATCH


def _body(idsr_hbm, tok_hbm, pos_hbm, out_hbm,
          idx_v, pos_v, rows_b, gsems, osems, psem):
    wid = lax.axis_index("s") * NC + lax.axis_index("c")
    s0 = wid * POS_PER_W

    pltpu.sync_copy(idsr_hbm.at[pl.ds(wid * IDX_PER_W, IDX_PER_W)], idx_v)

    def gather_cp(pc, bp, rb):
        off = pc * (2 * PC * 2) + bp * (2 * PC)
        return pltpu.make_async_copy(
            tok_hbm.at[idx_v.at[pl.ds(off, 2 * PC)]],
            rows_b.at[rb], gsems.at[rb])

    def out_cp(pc, bp, rb, h):
        return pltpu.make_async_copy(
            rows_b.at[rb, pl.ds(h * PC, PC)],
            out_hbm.at[pl.ds((2 * bp + h) * SEQ + s0 + pc * PC, PC)],
            osems.at[rb])

    def pos_cp(pc):
        return pltpu.make_async_copy(
            pos_hbm.at[pl.ds(s0 + pc * PC, PC)], pos_v, psem)

    def add_rows(rb):
        @plsc.parallel_loop(0, PC, unroll=2)
        def _(r):
            for j in range(HIDDEN // LANES):
                sl = pl.ds(j * LANES, LANES)
                x = pos_v[r, sl]
                plsc.addupdate(rows_b.at[rb, r, sl], x)
                plsc.addupdate(rows_b.at[rb, PC + r, sl], x)

    pos_cp(0).start()
    gather_cp(0, 0, 0).start()

    def cps(t):
        pc = lax.div(t, 2)
        bp = lax.rem(t, 2)
        rb = lax.rem(t, NRB)
        return pc, bp, rb

    def step(t, carry):
        pc, bp, rb = cps(t)
        @pl.when(t >= 2)
        def _():
            pc2, bp2, rb2 = cps(t - 2)
            for h in range(2):
                out_cp(pc2, bp2, rb2, h).wait()

        @pl.when(t + 1 < NSTEP)
        def _():
            pc1, bp1, rb1 = cps(t + 1)
            gather_cp(pc1, bp1, rb1).start()

        @pl.when(bp == 0)
        def _():
            pos_cp(pc).wait()

        gather_cp(pc, bp, rb).wait()
        add_rows(rb)

        @pl.when((bp == 1) & (pc + 1 < NPC))
        def _():
            pos_cp(pc + 1).start()

        for h in range(2):
            out_cp(pc, bp, rb, h).start()
        return carry

    lax.fori_loop(0, NSTEP, step, 0)

    for t in (NSTEP - 2, NSTEP - 1):
        pc, bp, rb = t // 2, t % 2, t % NRB
        for h in range(2):
            out_cp(pc, bp, rb, h).wait()


@jax.jit
def _embed(ids_re, token_table, pos_table):
    mesh = plsc.VectorSubcoreMesh(core_axis_name="c", subcore_axis_name="s")
    k = pl.kernel(
        _body,
        out_type=jax.ShapeDtypeStruct((BATCH * SEQ, HIDDEN), jnp.float32),
        mesh=mesh,
        scratch_types=[
            pltpu.VMEM((IDX_PER_W,), jnp.int32),
            pltpu.VMEM((PC, HIDDEN), jnp.float32),
            pltpu.VMEM((NRB, 2 * PC, HIDDEN), jnp.float32),
            pltpu.SemaphoreType.DMA((NRB,)),
            pltpu.SemaphoreType.DMA((NRB,)),
            pltpu.SemaphoreType.DMA,
        ],
    )
    return k(ids_re, token_table, pos_table)


def kernel(input_ids, token_table, pos_table):
    ids_re = (
        input_ids.astype(jnp.int32)
        .reshape(BATCH, SEQ // PC, PC)
        .transpose(1, 0, 2)
        .reshape(BATCH * SEQ)
    )
    out = _embed(ids_re, token_table, pos_table)
    return out.reshape(BATCH, SEQ, HIDDEN)
